# baseline (speedup 1.0000x reference)
import functools

import jax
import jax.numpy as jnp
from jax import lax
from jax.experimental import pallas as pl
from jax.experimental.pallas import tpu as pltpu
from jax.experimental.pallas import tpu_sc as plsc

NUM_CORES = 2
NUM_SUBCORES = 16
NW = NUM_CORES * NUM_SUBCORES
LANES = 16
CHUNK = 40
NBUF = 4


def _make_sc_gather_sum(B, KL, D):
    assert B % NW == 0 and D % LANES == 0
    bpw = B // NW
    cpb = KL // CHUNK
    assert cpb * CHUNK == KL and cpb % NBUF == 0 and CHUNK % 8 == 0
    nch = bpw * cpb
    ngr = nch // NBUF
    nj = D // LANES
    mesh = plsc.VectorSubcoreMesh(core_axis_name="c", subcore_axis_name="s")

    @functools.partial(
        pl.kernel,
        mesh=mesh,
        out_type=jax.ShapeDtypeStruct((B, D), jnp.float32),
        scratch_types=[
            pltpu.VMEM((bpw * KL,), jnp.int32),
            pltpu.VMEM((NBUF, CHUNK, D), jnp.float32),
            pltpu.VMEM((bpw, D), jnp.float32),
        ] + [pltpu.SemaphoreType.DMA] * NBUF,
    )
    def sc_kernel(idx_hbm, table_hbm, out_hbm, idx_v, bufs, outv, *sems):
        wid = lax.axis_index("s") * NUM_CORES + lax.axis_index("c")
        base_b = wid * bpw

        pltpu.sync_copy(idx_hbm.at[pl.ds(base_b * KL, bpw * KL)], idx_v)

        def start(c, i):
            pltpu.async_copy(
                table_hbm.at[idx_v.at[pl.ds(c * CHUNK, CHUNK)]],
                bufs.at[i], sems[i])

        def wait(i):
            pltpu.make_async_copy(
                table_hbm.at[idx_v.at[pl.ds(0, CHUNK)]],
                bufs.at[i], sems[i]).wait()

        def accum(i, acc):
            def rows4(r4, acc):
                accl = list(acc)
                for rr in range(4):
                    r = r4 * 4 + rr
                    for j in range(nj):
                        accl[j] = accl[j] + bufs[i, r, pl.ds(LANES * j, LANES)]
                return tuple(accl)
            return lax.fori_loop(0, CHUNK // 4, rows4, acc)

        for i in range(NBUF):
            start(i, i)

        zeros = tuple(jnp.zeros((LANES,), jnp.float32) for _ in range(nj))

        def group_body(g, acc):
            for i in range(NBUF):
                c = g * NBUF + i
                wait(i)
                acc = accum(i, acc)
                nxt = c + NBUF

                @pl.when(nxt < nch)
                def _():
                    start(nxt, i)

            is_b = (g % (cpb // NBUF)) == (cpb // NBUF - 1)
            bl = g // (cpb // NBUF)

            @pl.when(is_b)
            def _():
                for j in range(nj):
                    outv[bl, pl.ds(LANES * j, LANES)] = acc[j]

            return tuple(jnp.where(is_b, jnp.zeros((LANES,), jnp.float32), a)
                         for a in acc)

        lax.fori_loop(0, ngr, group_body, zeros)
        pltpu.sync_copy(outv, out_hbm.at[pl.ds(base_b, bpw)])

    return sc_kernel


def _proj_pool_kernel(sums_ref, maskf_ref, w_ref, b_ref, out_ref):
    cnt = jnp.sum(maskf_ref[...], axis=1, keepdims=True)
    denom = jnp.maximum(cnt, 1.0)
    pooled = sums_ref[...] / denom
    proj = lax.dot_general(pooled, w_ref[...], (((1,), (1,)), ((), ())),
                           preferred_element_type=jnp.float32)
    out_ref[...] = proj + b_ref[...] * (cnt / denom)


def kernel(ref_tokens, ref_mask, emb, W, b):
    B, K, L = ref_tokens.shape
    V, D = emb.shape[1], emb.shape[2]
    offs = (jnp.arange(K, dtype=jnp.int32) * V)[None, :, None]
    idx = (ref_tokens.astype(jnp.int32) + offs).reshape(-1)
    table = emb.reshape(K * V, D)

    sums = _make_sc_gather_sum(B, K * L, D)(idx, table)

    maskf = ref_mask.astype(jnp.float32)
    out = pl.pallas_call(
        _proj_pool_kernel,
        out_shape=jax.ShapeDtypeStruct((B, D), jnp.float32),
    )(sums, maskf, W, b.reshape(1, D))
    return out


# speedup vs baseline: 1.3245x; 1.3245x over previous
import functools

import jax
import jax.numpy as jnp
from jax import lax
from jax.experimental import pallas as pl
from jax.experimental.pallas import tpu as pltpu
from jax.experimental.pallas import tpu_sc as plsc

NUM_CORES = 2
NUM_SUBCORES = 16
NW = NUM_CORES * NUM_SUBCORES
LANES = 16
CHUNK = 100
NBUF = 4


def _make_sc_gather_sum(B, KL, D):
    assert B % NW == 0 and D % LANES == 0
    bpw = B // NW
    cpb = KL // CHUNK
    assert cpb * CHUNK == KL and cpb % NBUF == 0
    nch = bpw * cpb
    ngr = nch // NBUF
    nj = D // LANES
    mesh = plsc.VectorSubcoreMesh(core_axis_name="c", subcore_axis_name="s")

    @functools.partial(
        pl.kernel,
        mesh=mesh,
        out_type=jax.ShapeDtypeStruct((B, D), jnp.float32),
        scratch_types=[
            pltpu.VMEM((bpw * cpb, CHUNK), jnp.int32),
            pltpu.VMEM((NBUF, CHUNK, D), jnp.float32),
            pltpu.VMEM((bpw, D), jnp.float32),
        ] + [pltpu.SemaphoreType.DMA] * NBUF,
    )
    def sc_kernel(idx_hbm, table_hbm, out_hbm, idx_v, bufs, outv, *sems):
        wid = lax.axis_index("s") * NUM_CORES + lax.axis_index("c")
        base_b = wid * bpw

        pltpu.sync_copy(idx_hbm.at[pl.ds(base_b * cpb, bpw * cpb)], idx_v)

        def start(c, i):
            pltpu.async_copy(
                table_hbm.at[idx_v.at[c]],
                bufs.at[i], sems[i])

        def wait(i):
            pltpu.make_async_copy(
                table_hbm.at[idx_v.at[0]],
                bufs.at[i], sems[i]).wait()

        def accum(i, acc):
            def rows4(r4, acc):
                accl = list(acc)
                for rr in range(4):
                    r = r4 * 4 + rr
                    for j in range(nj):
                        accl[j] = accl[j] + bufs[i, r, pl.ds(LANES * j, LANES)]
                return tuple(accl)
            return lax.fori_loop(0, CHUNK // 4, rows4, acc)

        for i in range(NBUF):
            start(i, i)

        zeros = tuple(jnp.zeros((LANES,), jnp.float32) for _ in range(nj))

        def group_body(g, acc):
            for i in range(NBUF):
                c = g * NBUF + i
                wait(i)
                acc = accum(i, acc)
                nxt = c + NBUF

                @pl.when(nxt < nch)
                def _():
                    start(nxt, i)

            is_b = (g % (cpb // NBUF)) == (cpb // NBUF - 1)
            bl = g // (cpb // NBUF)

            @pl.when(is_b)
            def _():
                for j in range(nj):
                    outv[bl, pl.ds(LANES * j, LANES)] = acc[j]

            return tuple(jnp.where(is_b, jnp.zeros((LANES,), jnp.float32), a)
                         for a in acc)

        lax.fori_loop(0, ngr, group_body, zeros)
        pltpu.sync_copy(outv, out_hbm.at[pl.ds(base_b, bpw)])

    return sc_kernel


def _proj_pool_kernel(sums_ref, maskf_ref, w_ref, b_ref, out_ref):
    cnt = jnp.sum(maskf_ref[...], axis=1, keepdims=True)
    denom = jnp.maximum(cnt, 1.0)
    pooled = sums_ref[...] / denom
    proj = lax.dot_general(pooled, w_ref[...], (((1,), (1,)), ((), ())),
                           preferred_element_type=jnp.float32)
    out_ref[...] = proj + b_ref[...] * (cnt / denom)


def kernel(ref_tokens, ref_mask, emb, W, b):
    B, K, L = ref_tokens.shape
    V, D = emb.shape[1], emb.shape[2]
    offs = (jnp.arange(K, dtype=jnp.int32) * V)[None, :, None]
    idx = (ref_tokens.astype(jnp.int32) + offs).reshape(-1, CHUNK)
    table = emb.reshape(K * V, D)

    sums = _make_sc_gather_sum(B, K * L, D)(idx, table)

    maskf = ref_mask.astype(jnp.float32)
    out = pl.pallas_call(
        _proj_pool_kernel,
        out_shape=jax.ShapeDtypeStruct((B, D), jnp.float32),
    )(sums, maskf, W, b.reshape(1, D))
    return out
